# baseline (speedup 1.0000x reference)
import functools

import jax
import jax.numpy as jnp
from jax.experimental import pallas as pl


MEM_D = 32
NCLS = 10
NCLS_PAD = 16


def _gru_mlp_body(ms_ref, md_ref, msg_ref, t_ref, lus_ref, lud_ref, mlab_ref,
                  tw_ref, tb_ref,
                  wir_ref, wiz_ref, win_ref, whr_ref, whz_ref, whn_ref,
                  bir_ref, biz_ref, bin_ref, bhr_ref, bhz_ref, bhn_ref,
                  w1_ref, b1_ref, w2_ref, b2_ref,
                  ls_ref, ld_ref, lm_ref):
    ms = ms_ref[...]
    md = md_ref[...]
    msg = msg_ref[...]
    tf = t_ref[...].astype(jnp.float32)
    lus = lus_ref[...].astype(jnp.float32)
    lud = lud_ref[...].astype(jnp.float32)
    tw = tw_ref[...]
    tb = tb_ref[...]

    def dot_t(a, w):
        return jax.lax.dot_general(a, w, (((1,), (1,)), ((), ())),
                                   preferred_element_type=jnp.float32)

    def gru(h, other, rel):
        tenc = jnp.cos(rel * tw + tb)
        x = jnp.concatenate([h, other, msg, tenc], axis=1)
        r = jax.nn.sigmoid(dot_t(x, wir_ref[...]) + bir_ref[...]
                           + dot_t(h, whr_ref[...]) + bhr_ref[...])
        z = jax.nn.sigmoid(dot_t(x, wiz_ref[...]) + biz_ref[...]
                           + dot_t(h, whz_ref[...]) + bhz_ref[...])
        n = jnp.tanh(dot_t(x, win_ref[...]) + bin_ref[...]
                     + r * (dot_t(h, whn_ref[...]) + bhn_ref[...]))
        return (1.0 - z) * n + z * h

    nh_s = gru(ms, md, tf - lus)
    nh_d = gru(md, ms, tf - lud)

    def mlp(h):
        h1 = jax.nn.relu(dot_t(h, w1_ref[...]) + b1_ref[...])
        return dot_t(h1, w2_ref[...]) + b2_ref[...]

    ls_ref[...] = mlp(nh_s)
    ld_ref[...] = mlp(nh_d)
    lm_ref[...] = mlp(mlab_ref[...])


def _gru_mlp(ms, md, msg, t, lus, lud, mlab, tw, tb,
             wir, wiz, win, whr, whz, whn, bir, biz, bin_, bhr, bhz, bhn,
             w1, b1, w2, b2):
    B = ms.shape[0]
    bs = 2048
    grid = (B // bs,)
    row = lambda i: (i, 0)
    full = lambda shape: pl.BlockSpec(shape, lambda i: (0, 0))
    rspec = lambda d: pl.BlockSpec((bs, d), row)
    in_specs = [
        rspec(MEM_D), rspec(MEM_D), rspec(MEM_D), rspec(1), rspec(1), rspec(1),
        rspec(MEM_D),
        full((1, MEM_D)), full((1, MEM_D)),
        full((MEM_D, 4 * MEM_D)), full((MEM_D, 4 * MEM_D)), full((MEM_D, 4 * MEM_D)),
        full((MEM_D, MEM_D)), full((MEM_D, MEM_D)), full((MEM_D, MEM_D)),
        full((1, MEM_D)), full((1, MEM_D)), full((1, MEM_D)),
        full((1, MEM_D)), full((1, MEM_D)), full((1, MEM_D)),
        full((MEM_D, MEM_D)), full((1, MEM_D)),
        full((NCLS_PAD, MEM_D)), full((1, NCLS_PAD)),
    ]
    out_specs = [rspec(NCLS_PAD)] * 3
    out_shape = [jax.ShapeDtypeStruct((B, NCLS_PAD), jnp.float32)] * 3
    return pl.pallas_call(
        _gru_mlp_body,
        grid=grid,
        in_specs=in_specs,
        out_specs=out_specs,
        out_shape=out_shape,
    )(ms, md, msg, t, lus, lud, mlab, tw, tb,
      wir, wiz, win, whr, whz, whn, bir, biz, bin_, bhr, bhz, bhn,
      w1, b1, w2, b2)


def kernel(src, dst, t, msg, label_nodes, current_time, memory, last_update,
           time_w, time_b, W_ih, W_hh, b_ih, b_hh, W1, b1, W2, b2):
    B = src.shape[0]
    num_nodes = memory.shape[0]
    D = MEM_D

    iota1 = jnp.arange(1, B + 1, dtype=jnp.int32)
    A = jnp.zeros((num_nodes,), jnp.int32).at[src].max(iota1)
    Dd = jnp.zeros((num_nodes,), jnp.int32).at[dst].max(iota1)
    ia = A[label_nodes]
    jd = Dd[label_nodes]
    ts = t[jnp.maximum(ia - 1, 0)]
    td = t[jnp.maximum(jd - 1, 0)]
    use_dst = (jd > 0) & ((ia == 0) | (td >= ts))
    use_src = (ia > 0) & ~use_dst
    qpos = jnp.arange(B, dtype=jnp.int32)
    sel = jnp.where(use_dst, B + jd - 1,
                    jnp.where(use_src, ia - 1, 2 * B + qpos))

    ms = memory[src]
    md = memory[dst]
    lus = last_update[src]
    lud = last_update[dst]
    mlab = memory[label_nodes]

    wi = W_ih
    wh = W_hh
    wir, wiz, win = wi[:D], wi[D:2 * D], wi[2 * D:]
    whr, whz, whn = wh[:D], wh[D:2 * D], wh[2 * D:]
    bir, biz, bin_ = b_ih[:D], b_ih[D:2 * D], b_ih[2 * D:]
    bhr, bhz, bhn = b_hh[:D], b_hh[D:2 * D], b_hh[2 * D:]
    w2p = jnp.zeros((NCLS_PAD, D), jnp.float32).at[:NCLS].set(W2)
    b2p = jnp.zeros((NCLS_PAD,), jnp.float32).at[:NCLS].set(b2)

    r2 = lambda v: v.reshape(1, -1)
    c1 = lambda v: v.reshape(-1, 1)
    ls, ld, lm = _gru_mlp(
        ms, md, msg, c1(t), c1(lus), c1(lud), mlab,
        r2(time_w), r2(time_b),
        wir, wiz, win, whr, whz, whn,
        r2(bir), r2(biz), r2(bin_), r2(bhr), r2(bhz), r2(bhn),
        W1, r2(b1), w2p, r2(b2p))

    L = jnp.concatenate([ls, ld, lm], axis=0)
    return L[sel, :NCLS]


# speedup vs baseline: 88.5567x; 88.5567x over previous
import functools

import jax
import jax.numpy as jnp
from jax import lax
from jax.experimental import pallas as pl
from jax.experimental.pallas import tpu as pltpu
from jax.experimental.pallas import tpu_sc as plsc


MEM_D = 32
NCLS = 10
NCLS_PAD = 16
B = 16384
NODE_CAP = 1 << 20
NC = 2
NS = 16
NW = NC * NS
BW = B // NW
OWN = NODE_CAP // NW
OWN_SHIFT = 15


def _mesh():
    return plsc.VectorSubcoreMesh(core_axis_name="c", subcore_axis_name="s")


def _wid():
    return lax.axis_index("s") * NC + lax.axis_index("c")


def _sc1_body(src_hbm, dst_hbm, lab_hbm, mem_hbm, lu_hbm,
              ms_hbm, md_hbm, mlab_hbm, lus_hbm, lud_hbm, a_hbm, d_hbm,
              idx_v, rows_v, ele_v, ids_v, ta_v, td_v, bnc_v, sem):
    wid = _wid()
    base = wid * BW
    iota = lax.iota(jnp.int32, 16)

    def gather_rows(src_idx_hbm, out_rows_hbm):
        pltpu.sync_copy(src_idx_hbm.at[pl.ds(base, BW)], idx_v)
        for j in range(BW // 128):
            pltpu.async_copy(
                mem_hbm.at[idx_v.at[pl.ds(j * 128, 128)]],
                rows_v.at[pl.ds(j * 128, 128)], sem).wait()
        pltpu.sync_copy(rows_v, out_rows_hbm.at[pl.ds(base, BW)])

    def gather_lu(out_hbm):
        for j in range(BW // 128):
            pltpu.async_copy(
                lu_hbm.at[idx_v.at[pl.ds(j * 128, 128)]],
                ele_v.at[pl.ds(j * 128, 128)], sem).wait()
        pltpu.sync_copy(ele_v, out_hbm.at[pl.ds(base, BW)])

    gather_rows(src_hbm, ms_hbm)
    gather_lu(lus_hbm)
    gather_rows(dst_hbm, md_hbm)
    gather_lu(lud_hbm)
    gather_rows(lab_hbm, mlab_hbm)

    zero = jnp.zeros((16,), jnp.int32)

    def init_body(i, carry):
        ta_v[pl.ds(i * 16, 16)] = zero
        td_v[pl.ds(i * 16, 16)] = zero
        return carry

    lax.fori_loop(0, OWN // 16, init_body, 0)

    bnc_v[pl.ds(16, 16)] = jnp.full((16,), -1, jnp.int32)

    def build(table_ref):
        def body(c, carry):
            ids = ids_v[pl.ds(c * 16, 16)]
            key = ids * 16 + iota
            ks, _ = plsc.sort_key_val(key, key)
            bnc_v[pl.ds(0, 16)] = ks
            nxt = bnc_v[pl.ds(1, 16)]
            id_s = lax.shift_right_arithmetic(ks, 4)
            id_n = lax.shift_right_arithmetic(nxt, 4)
            seg_end = id_s != id_n
            owned = lax.shift_right_arithmetic(id_s, OWN_SHIFT) == wid
            m = seg_end & owned
            e_val = c * 16 + (ks & 15) + 1
            loc = id_s & (OWN - 1)
            plsc.store_scatter(table_ref, [loc], e_val, mask=m)
            return carry

        lax.fori_loop(0, B // 16, body, 0)

    pltpu.sync_copy(src_hbm, ids_v)
    build(ta_v)
    pltpu.sync_copy(dst_hbm, ids_v)
    build(td_v)

    pltpu.sync_copy(ta_v, a_hbm.at[pl.ds(wid * OWN, OWN)])
    pltpu.sync_copy(td_v, d_hbm.at[pl.ds(wid * OWN, OWN)])


def _sc1(src, dst, lab, memory, last_update):
    out_type = [
        jax.ShapeDtypeStruct((B, MEM_D), jnp.float32),
        jax.ShapeDtypeStruct((B, MEM_D), jnp.float32),
        jax.ShapeDtypeStruct((B, MEM_D), jnp.float32),
        jax.ShapeDtypeStruct((B,), jnp.int32),
        jax.ShapeDtypeStruct((B,), jnp.int32),
        jax.ShapeDtypeStruct((NODE_CAP,), jnp.int32),
        jax.ShapeDtypeStruct((NODE_CAP,), jnp.int32),
    ]
    scratch = [
        pltpu.VMEM((BW,), jnp.int32),
        pltpu.VMEM((BW, MEM_D), jnp.float32),
        pltpu.VMEM((BW,), jnp.int32),
        pltpu.VMEM((B,), jnp.int32),
        pltpu.VMEM((OWN,), jnp.int32),
        pltpu.VMEM((OWN,), jnp.int32),
        pltpu.VMEM((32,), jnp.int32),
        pltpu.SemaphoreType.DMA,
    ]
    fn = pl.kernel(_sc1_body, mesh=_mesh(), out_type=out_type,
                   scratch_types=scratch,
                   compiler_params=pltpu.CompilerParams(
                       needs_layout_passes=False,
                       use_tc_tiling_on_sc=False))
    return fn(src, dst, lab, memory, last_update)


def _sc2_body(lab_hbm, t_hbm, a_hbm, d_hbm, l_hbm, out_hbm,
              lab_v, ia_v, jd_v, im_v, jm_v, ts_v, td_v, sel_v, rows_v, sem):
    wid = _wid()
    base = wid * BW
    iota = lax.iota(jnp.int32, 16)

    pltpu.sync_copy(lab_hbm.at[pl.ds(base, BW)], lab_v)
    for j in range(BW // 128):
        pltpu.async_copy(a_hbm.at[lab_v.at[pl.ds(j * 128, 128)]],
                         ia_v.at[pl.ds(j * 128, 128)], sem).wait()
        pltpu.async_copy(d_hbm.at[lab_v.at[pl.ds(j * 128, 128)]],
                         jd_v.at[pl.ds(j * 128, 128)], sem).wait()

    def clamp_body(c, carry):
        ia = ia_v[pl.ds(c * 16, 16)]
        jd = jd_v[pl.ds(c * 16, 16)]
        im_v[pl.ds(c * 16, 16)] = jnp.maximum(ia - 1, 0)
        jm_v[pl.ds(c * 16, 16)] = jnp.maximum(jd - 1, 0)
        return carry

    lax.fori_loop(0, BW // 16, clamp_body, 0)

    for j in range(BW // 128):
        pltpu.async_copy(t_hbm.at[im_v.at[pl.ds(j * 128, 128)]],
                         ts_v.at[pl.ds(j * 128, 128)], sem).wait()
        pltpu.async_copy(t_hbm.at[jm_v.at[pl.ds(j * 128, 128)]],
                         td_v.at[pl.ds(j * 128, 128)], sem).wait()

    def sel_body(c, carry):
        ia = ia_v[pl.ds(c * 16, 16)]
        jd = jd_v[pl.ds(c * 16, 16)]
        ts = ts_v[pl.ds(c * 16, 16)]
        td = td_v[pl.ds(c * 16, 16)]
        use_dst = (jd > 0) & ((ia == 0) | (td >= ts))
        use_src = (ia > 0) & (~use_dst)
        qpos = base + c * 16 + iota
        sel = jnp.where(use_dst, B + jd - 1,
                        jnp.where(use_src, ia - 1, 2 * B + qpos))
        sel_v[pl.ds(c * 16, 16)] = sel
        return carry

    lax.fori_loop(0, BW // 16, sel_body, 0)

    for j in range(BW // 128):
        pltpu.async_copy(l_hbm.at[sel_v.at[pl.ds(j * 128, 128)]],
                         rows_v.at[pl.ds(j * 128, 128)], sem).wait()
    pltpu.sync_copy(rows_v, out_hbm.at[pl.ds(base, BW)])


def _sc2(lab, t, a_tab, d_tab, l_rows):
    scratch = [
        pltpu.VMEM((BW,), jnp.int32),
        pltpu.VMEM((BW,), jnp.int32),
        pltpu.VMEM((BW,), jnp.int32),
        pltpu.VMEM((BW,), jnp.int32),
        pltpu.VMEM((BW,), jnp.int32),
        pltpu.VMEM((BW,), jnp.int32),
        pltpu.VMEM((BW,), jnp.int32),
        pltpu.VMEM((BW,), jnp.int32),
        pltpu.VMEM((BW, NCLS_PAD), jnp.float32),
        pltpu.SemaphoreType.DMA,
    ]
    fn = pl.kernel(_sc2_body, mesh=_mesh(),
                   out_type=jax.ShapeDtypeStruct((B, NCLS_PAD), jnp.float32),
                   scratch_types=scratch,
                   compiler_params=pltpu.CompilerParams(
                       needs_layout_passes=False,
                       use_tc_tiling_on_sc=False))
    return fn(lab, t, a_tab, d_tab, l_rows)


def _gru_mlp_body(ms_ref, md_ref, msg_ref, t_ref, lus_ref, lud_ref, mlab_ref,
                  tw_ref, tb_ref,
                  wir_ref, wiz_ref, win_ref, whr_ref, whz_ref, whn_ref,
                  bir_ref, biz_ref, bin_ref, bhr_ref, bhz_ref, bhn_ref,
                  w1_ref, b1_ref, w2_ref, b2_ref,
                  ls_ref, ld_ref, lm_ref):
    ms = ms_ref[...]
    md = md_ref[...]
    msg = msg_ref[...]
    tf = t_ref[...].astype(jnp.float32)
    lus = lus_ref[...].astype(jnp.float32)
    lud = lud_ref[...].astype(jnp.float32)
    tw = tw_ref[...]
    tb = tb_ref[...]

    def dot_t(a, w):
        return lax.dot_general(a, w, (((1,), (1,)), ((), ())),
                               preferred_element_type=jnp.float32)

    def gru(h, other, rel):
        tenc = jnp.cos(rel * tw + tb)
        x = jnp.concatenate([h, other, msg, tenc], axis=1)
        r = jax.nn.sigmoid(dot_t(x, wir_ref[...]) + bir_ref[...]
                           + dot_t(h, whr_ref[...]) + bhr_ref[...])
        z = jax.nn.sigmoid(dot_t(x, wiz_ref[...]) + biz_ref[...]
                           + dot_t(h, whz_ref[...]) + bhz_ref[...])
        n = jnp.tanh(dot_t(x, win_ref[...]) + bin_ref[...]
                     + r * (dot_t(h, whn_ref[...]) + bhn_ref[...]))
        return (1.0 - z) * n + z * h

    nh_s = gru(ms, md, tf - lus)
    nh_d = gru(md, ms, tf - lud)

    def mlp(h):
        h1 = jax.nn.relu(dot_t(h, w1_ref[...]) + b1_ref[...])
        return dot_t(h1, w2_ref[...]) + b2_ref[...]

    ls_ref[...] = mlp(nh_s)
    ld_ref[...] = mlp(nh_d)
    lm_ref[...] = mlp(mlab_ref[...])


def _gru_mlp(ms, md, msg, t, lus, lud, mlab, tw, tb,
             wir, wiz, win, whr, whz, whn, bir, biz, bin_, bhr, bhz, bhn,
             w1, b1, w2, b2):
    bs = 2048
    grid = (B // bs,)
    row = lambda i: (i, 0)
    full = lambda shape: pl.BlockSpec(shape, lambda i: (0, 0))
    rspec = lambda d: pl.BlockSpec((bs, d), row)
    in_specs = [
        rspec(MEM_D), rspec(MEM_D), rspec(MEM_D), rspec(1), rspec(1), rspec(1),
        rspec(MEM_D),
        full((1, MEM_D)), full((1, MEM_D)),
        full((MEM_D, 4 * MEM_D)), full((MEM_D, 4 * MEM_D)), full((MEM_D, 4 * MEM_D)),
        full((MEM_D, MEM_D)), full((MEM_D, MEM_D)), full((MEM_D, MEM_D)),
        full((1, MEM_D)), full((1, MEM_D)), full((1, MEM_D)),
        full((1, MEM_D)), full((1, MEM_D)), full((1, MEM_D)),
        full((MEM_D, MEM_D)), full((1, MEM_D)),
        full((NCLS_PAD, MEM_D)), full((1, NCLS_PAD)),
    ]
    out_specs = [rspec(NCLS_PAD)] * 3
    out_shape = [jax.ShapeDtypeStruct((B, NCLS_PAD), jnp.float32)] * 3
    return pl.pallas_call(
        _gru_mlp_body,
        grid=grid,
        in_specs=in_specs,
        out_specs=out_specs,
        out_shape=out_shape,
    )(ms, md, msg, t, lus, lud, mlab, tw, tb,
      wir, wiz, win, whr, whz, whn, bir, biz, bin_, bhr, bhz, bhn,
      w1, b1, w2, b2)


def kernel(src, dst, t, msg, label_nodes, current_time, memory, last_update,
           time_w, time_b, W_ih, W_hh, b_ih, b_hh, W1, b1, W2, b2):
    D = MEM_D

    ms, md, mlab, lus, lud, a_tab, d_tab = _sc1(
        src, dst, label_nodes, memory, last_update)

    wir, wiz, win = W_ih[:D], W_ih[D:2 * D], W_ih[2 * D:]
    whr, whz, whn = W_hh[:D], W_hh[D:2 * D], W_hh[2 * D:]
    bir, biz, bin_ = b_ih[:D], b_ih[D:2 * D], b_ih[2 * D:]
    bhr, bhz, bhn = b_hh[:D], b_hh[D:2 * D], b_hh[2 * D:]
    w2p = jnp.zeros((NCLS_PAD, D), jnp.float32).at[:NCLS].set(W2)
    b2p = jnp.zeros((NCLS_PAD,), jnp.float32).at[:NCLS].set(b2)

    r2 = lambda v: v.reshape(1, -1)
    c1 = lambda v: v.reshape(-1, 1)
    ls, ld, lm = _gru_mlp(
        ms, md, msg, c1(t), c1(lus), c1(lud), mlab,
        r2(time_w), r2(time_b),
        wir, wiz, win, whr, whz, whn,
        r2(bir), r2(biz), r2(bin_), r2(bhr), r2(bhz), r2(bhn),
        W1, r2(b1), w2p, r2(b2p))

    l_rows = jnp.concatenate([ls, ld, lm], axis=0)
    out = _sc2(label_nodes, t, a_tab, d_tab, l_rows)
    return out[:, :NCLS]


# speedup vs baseline: 89.4948x; 1.0106x over previous
import functools

import jax
import jax.numpy as jnp
from jax import lax
from jax.experimental import pallas as pl
from jax.experimental.pallas import tpu as pltpu
from jax.experimental.pallas import tpu_sc as plsc


MEM_D = 32
NCLS = 10
NCLS_PAD = 16
B = 16384
NNODES = 1000000
NODE_CAP = 1 << 20
NC = 2
NS = 16
NW = NC * NS
BW = B // NW
OWN = NODE_CAP // NW
OWN_SHIFT = 15


def _mesh():
    return plsc.VectorSubcoreMesh(core_axis_name="c", subcore_axis_name="s")


def _wid():
    return lax.axis_index("s") * NC + lax.axis_index("c")


_SC_PARAMS = pltpu.CompilerParams(needs_layout_passes=False,
                                  use_tc_tiling_on_sc=False)


def _sca_body(src_hbm, dst_hbm, lab_hbm, mem_hbm, lu_hbm,
              ms_hbm, md_hbm, mlab_hbm, lus_hbm, lud_hbm, a_hbm, d_hbm,
              own_v, row_v, ele_v, ids_v, ta_v, td_v, bnc_v, sem):
    wid = _wid()
    base = wid * BW
    iota = lax.iota(jnp.int32, 16)

    def fire_rows():
        for j in range(BW // 128):
            pltpu.async_copy(
                mem_hbm.at[own_v.at[pl.ds(j * 128, 128)]],
                row_v.at[pl.ds(j * 128, 128)], sem)

    def drain_rows(out_hbm):
        for j in range(BW // 128):
            pltpu.make_async_copy(
                mem_hbm.at[own_v.at[pl.ds(j * 128, 128)]],
                row_v.at[pl.ds(j * 128, 128)], sem).wait()
        pltpu.sync_copy(row_v, out_hbm.at[pl.ds(base, BW)])

    def gather_lu(out_hbm):
        for j in range(BW // 128):
            pltpu.async_copy(
                lu_hbm.at[own_v.at[pl.ds(j * 128, 128)]],
                ele_v.at[pl.ds(j * 128, 128)], sem).wait()
        pltpu.sync_copy(ele_v, out_hbm.at[pl.ds(base, BW)])

    zero = jnp.zeros((16,), jnp.int32)

    def init_body(i, carry):
        ta_v[pl.ds(i * 16, 16)] = zero
        td_v[pl.ds(i * 16, 16)] = zero
        return carry

    def build(table_ref):
        def body(c, carry):
            for u in range(4):
                cc = c * 4 + u
                ids = ids_v[pl.ds(cc * 16, 16)]
                key = ids * 16 + iota
                ks, _ = plsc.sort_key_val(key, key)
                bnc_v[pl.ds(0, 16)] = ks
                nxt = bnc_v[pl.ds(1, 16)]
                id_s = lax.shift_right_arithmetic(ks, 4)
                id_n = lax.shift_right_arithmetic(nxt, 4)
                seg_end = id_s != id_n
                owned = lax.shift_right_arithmetic(id_s, OWN_SHIFT) == wid
                m = seg_end & owned
                e_val = cc * 16 + (ks & 15) + 1
                loc = id_s & (OWN - 1)
                plsc.store_scatter(table_ref, [loc], e_val, mask=m)
            return carry

        lax.fori_loop(0, B // 64, body, 0)

    bnc_v[pl.ds(16, 16)] = jnp.full((16,), -1, jnp.int32)

    pltpu.sync_copy(src_hbm.at[pl.ds(base, BW)], own_v)
    fire_rows()
    lax.fori_loop(0, OWN // 16, init_body, 0)
    pltpu.sync_copy(src_hbm, ids_v)
    build(ta_v)
    drain_rows(ms_hbm)
    gather_lu(lus_hbm)

    pltpu.sync_copy(dst_hbm.at[pl.ds(base, BW)], own_v)
    fire_rows()
    pltpu.sync_copy(dst_hbm, ids_v)
    build(td_v)
    drain_rows(md_hbm)
    gather_lu(lud_hbm)

    pltpu.sync_copy(lab_hbm.at[pl.ds(base, BW)], own_v)
    fire_rows()
    pltpu.sync_copy(ta_v, a_hbm.at[pl.ds(wid * OWN, OWN)])
    pltpu.sync_copy(td_v, d_hbm.at[pl.ds(wid * OWN, OWN)])
    drain_rows(mlab_hbm)


def _sca(src, dst, lab, mem, last_update):
    out_type = [
        jax.ShapeDtypeStruct((B, MEM_D), jnp.float32),
        jax.ShapeDtypeStruct((B, MEM_D), jnp.float32),
        jax.ShapeDtypeStruct((B, MEM_D), jnp.float32),
        jax.ShapeDtypeStruct((B,), jnp.int32),
        jax.ShapeDtypeStruct((B,), jnp.int32),
        jax.ShapeDtypeStruct((NODE_CAP,), jnp.int32),
        jax.ShapeDtypeStruct((NODE_CAP,), jnp.int32),
    ]
    scratch = [
        pltpu.VMEM((BW,), jnp.int32),
        pltpu.VMEM((BW, MEM_D), jnp.float32),
        pltpu.VMEM((BW,), jnp.int32),
        pltpu.VMEM((B,), jnp.int32),
        pltpu.VMEM((OWN,), jnp.int32),
        pltpu.VMEM((OWN,), jnp.int32),
        pltpu.VMEM((32,), jnp.int32),
        pltpu.SemaphoreType.DMA,
    ]
    fn = pl.kernel(_sca_body, mesh=_mesh(), out_type=out_type,
                   scratch_types=scratch, compiler_params=_SC_PARAMS)
    return fn(src, dst, lab, mem, last_update)


def _scb_body(lab_hbm, t_hbm, a_hbm, d_hbm, l_hbm, out_hbm,
              lab_v, ia_v, jd_v, im_v, jm_v, ts_v, td_v, sel_v, rows_v, sem):
    wid = _wid()
    base = wid * BW
    iota = lax.iota(jnp.int32, 16)

    pltpu.sync_copy(lab_hbm.at[pl.ds(base, BW)], lab_v)
    for j in range(BW // 128):
        pltpu.async_copy(a_hbm.at[lab_v.at[pl.ds(j * 128, 128)]],
                         ia_v.at[pl.ds(j * 128, 128)], sem).wait()
        pltpu.async_copy(d_hbm.at[lab_v.at[pl.ds(j * 128, 128)]],
                         jd_v.at[pl.ds(j * 128, 128)], sem).wait()

    def clamp_body(c, carry):
        ia = ia_v[pl.ds(c * 16, 16)]
        jd = jd_v[pl.ds(c * 16, 16)]
        im_v[pl.ds(c * 16, 16)] = jnp.maximum(ia - 1, 0)
        jm_v[pl.ds(c * 16, 16)] = jnp.maximum(jd - 1, 0)
        return carry

    lax.fori_loop(0, BW // 16, clamp_body, 0)

    for j in range(BW // 128):
        pltpu.async_copy(t_hbm.at[im_v.at[pl.ds(j * 128, 128)]],
                         ts_v.at[pl.ds(j * 128, 128)], sem).wait()
        pltpu.async_copy(t_hbm.at[jm_v.at[pl.ds(j * 128, 128)]],
                         td_v.at[pl.ds(j * 128, 128)], sem).wait()

    def sel_body(c, carry):
        ia = ia_v[pl.ds(c * 16, 16)]
        jd = jd_v[pl.ds(c * 16, 16)]
        ts = ts_v[pl.ds(c * 16, 16)]
        td = td_v[pl.ds(c * 16, 16)]
        use_dst = (jd > 0) & ((ia == 0) | (td >= ts))
        use_src = (ia > 0) & (~use_dst)
        qpos = base + c * 16 + iota
        sel = jnp.where(use_dst, B + jd - 1,
                        jnp.where(use_src, ia - 1, 2 * B + qpos))
        sel_v[pl.ds(c * 16, 16)] = sel
        return carry

    lax.fori_loop(0, BW // 16, sel_body, 0)

    for j in range(BW // 128):
        pltpu.async_copy(l_hbm.at[sel_v.at[pl.ds(j * 128, 128)]],
                         rows_v.at[pl.ds(j * 128, 128)], sem).wait()
    pltpu.sync_copy(rows_v, out_hbm.at[pl.ds(base, BW)])


def _scb(lab, t, a_tab, d_tab, l_rows):
    scratch = [
        pltpu.VMEM((BW,), jnp.int32),
        pltpu.VMEM((BW,), jnp.int32),
        pltpu.VMEM((BW,), jnp.int32),
        pltpu.VMEM((BW,), jnp.int32),
        pltpu.VMEM((BW,), jnp.int32),
        pltpu.VMEM((BW,), jnp.int32),
        pltpu.VMEM((BW,), jnp.int32),
        pltpu.VMEM((BW,), jnp.int32),
        pltpu.VMEM((BW, NCLS_PAD), jnp.float32),
        pltpu.SemaphoreType.DMA,
    ]
    fn = pl.kernel(_scb_body, mesh=_mesh(),
                   out_type=jax.ShapeDtypeStruct((B, NCLS_PAD), jnp.float32),
                   scratch_types=scratch, compiler_params=_SC_PARAMS)
    return fn(lab, t, a_tab, d_tab, l_rows)


def _gru_mlp_body(ms_ref, md_ref, msg_ref, t_ref, lus_ref, lud_ref, mlab_ref,
                  tw_ref, tb_ref,
                  wir_ref, wiz_ref, win_ref, whr_ref, whz_ref, whn_ref,
                  bir_ref, biz_ref, bin_ref, bhr_ref, bhz_ref, bhn_ref,
                  w1_ref, b1_ref, w2_ref, b2_ref,
                  ls_ref, ld_ref, lm_ref):
    ms = ms_ref[...]
    md = md_ref[...]
    msg = msg_ref[...]
    tf = t_ref[...].astype(jnp.float32)
    lus = lus_ref[...].astype(jnp.float32)
    lud = lud_ref[...].astype(jnp.float32)
    tw = tw_ref[...]
    tb = tb_ref[...]

    def dot_t(a, w):
        return lax.dot_general(a, w, (((1,), (1,)), ((), ())),
                               preferred_element_type=jnp.float32)

    def gru(h, other, rel):
        tenc = jnp.cos(rel * tw + tb)
        x = jnp.concatenate([h, other, msg, tenc], axis=1)
        r = jax.nn.sigmoid(dot_t(x, wir_ref[...]) + bir_ref[...]
                           + dot_t(h, whr_ref[...]) + bhr_ref[...])
        z = jax.nn.sigmoid(dot_t(x, wiz_ref[...]) + biz_ref[...]
                           + dot_t(h, whz_ref[...]) + bhz_ref[...])
        n = jnp.tanh(dot_t(x, win_ref[...]) + bin_ref[...]
                     + r * (dot_t(h, whn_ref[...]) + bhn_ref[...]))
        return (1.0 - z) * n + z * h

    nh_s = gru(ms, md, tf - lus)
    nh_d = gru(md, ms, tf - lud)

    def mlp(h):
        h1 = jax.nn.relu(dot_t(h, w1_ref[...]) + b1_ref[...])
        return dot_t(h1, w2_ref[...]) + b2_ref[...]

    ls_ref[...] = mlp(nh_s)
    ld_ref[...] = mlp(nh_d)
    lm_ref[...] = mlp(mlab_ref[...])


def _gru_mlp(ms, md, msg, t, lus, lud, mlab, tw, tb,
             wir, wiz, win, whr, whz, whn, bir, biz, bin_, bhr, bhz, bhn,
             w1, b1, w2, b2):
    bs = 2048
    grid = (B // bs,)
    row = lambda i: (i, 0)
    full = lambda shape: pl.BlockSpec(shape, lambda i: (0, 0))
    rspec = lambda d: pl.BlockSpec((bs, d), row)
    in_specs = [
        rspec(MEM_D), rspec(MEM_D), rspec(MEM_D), rspec(1), rspec(1), rspec(1),
        rspec(MEM_D),
        full((1, MEM_D)), full((1, MEM_D)),
        full((MEM_D, 4 * MEM_D)), full((MEM_D, 4 * MEM_D)), full((MEM_D, 4 * MEM_D)),
        full((MEM_D, MEM_D)), full((MEM_D, MEM_D)), full((MEM_D, MEM_D)),
        full((1, MEM_D)), full((1, MEM_D)), full((1, MEM_D)),
        full((1, MEM_D)), full((1, MEM_D)), full((1, MEM_D)),
        full((MEM_D, MEM_D)), full((1, MEM_D)),
        full((NCLS_PAD, MEM_D)), full((1, NCLS_PAD)),
    ]
    out_specs = [rspec(NCLS_PAD)] * 3
    out_shape = [jax.ShapeDtypeStruct((B, NCLS_PAD), jnp.float32)] * 3
    return pl.pallas_call(
        _gru_mlp_body,
        grid=grid,
        in_specs=in_specs,
        out_specs=out_specs,
        out_shape=out_shape,
    )(ms, md, msg, t, lus, lud, mlab, tw, tb,
      wir, wiz, win, whr, whz, whn, bir, biz, bin_, bhr, bhz, bhn,
      w1, b1, w2, b2)


def kernel(src, dst, t, msg, label_nodes, current_time, memory, last_update,
           time_w, time_b, W_ih, W_hh, b_ih, b_hh, W1, b1, W2, b2):
    D = MEM_D

    ms, md, mlab, lus, lud, a_tab, d_tab = _sca(
        src, dst, label_nodes, memory, last_update)

    wir, wiz, win = W_ih[:D], W_ih[D:2 * D], W_ih[2 * D:]
    whr, whz, whn = W_hh[:D], W_hh[D:2 * D], W_hh[2 * D:]
    bir, biz, bin_ = b_ih[:D], b_ih[D:2 * D], b_ih[2 * D:]
    bhr, bhz, bhn = b_hh[:D], b_hh[D:2 * D], b_hh[2 * D:]
    w2p = jnp.zeros((NCLS_PAD, D), jnp.float32).at[:NCLS].set(W2)
    b2p = jnp.zeros((NCLS_PAD,), jnp.float32).at[:NCLS].set(b2)

    r2 = lambda v: v.reshape(1, -1)
    c1 = lambda v: v.reshape(-1, 1)
    ls, ld, lm = _gru_mlp(
        ms, md, msg, c1(t), c1(lus), c1(lud), mlab,
        r2(time_w), r2(time_b),
        wir, wiz, win, whr, whz, whn,
        r2(bir), r2(biz), r2(bin_), r2(bhr), r2(bhz), r2(bhn),
        W1, r2(b1), w2p, r2(b2p))

    l_rows = jnp.concatenate([ls, ld, lm], axis=0)
    out = _scb(label_nodes, t, a_tab, d_tab, l_rows)
    return out[:, :NCLS]
